# Initial kernel scaffold; baseline (speedup 1.0000x reference)
import jax
import jax.numpy as jnp
from jax.experimental import pallas as pl


def kernel(user_emb, item_emb, edge_index_rate, edge_index_social, pos_edge_index, neg_edge_index):
    raise NotImplementedError("write your pallas kernel here")


# speedup vs baseline: 5.8843x; 5.8843x over previous
import functools

import jax
import jax.numpy as jnp
from jax import lax
from jax.experimental import pallas as pl
from jax.experimental.pallas import tpu as pltpu
from jax.experimental.pallas import tpu_sc as plsc

N = 25000
D = 64
NPAD = 25088
RPT = NPAD // 16
TRASH = NPAD - 1

P = 100000

CHUNK = 128
G = 7
RATE_PT = 50176
CR = RATE_PT // CHUNK
GR = CR // G
SOC_PT = 25088
CS = SOC_PT // CHUNK
GS = CS // G
SCORE_PT = 3200
CP = SCORE_PT // CHUNK

_mesh = plsc.VectorSubcoreMesh(
    core_axis_name="c", subcore_axis_name="s", num_cores=2, num_subcores=16)

_f32 = jnp.float32
_i32 = jnp.int32


@functools.partial(
    pl.kernel,
    out_type=tuple(jax.ShapeDtypeStruct((NPAD, 16), _f32) for _ in range(4)),
    mesh=_mesh,
    compiler_params=pltpu.CompilerParams(use_tc_tiling_on_sc=False),
    scratch_types=[
        pltpu.VMEM((G, CHUNK), _i32),
        pltpu.VMEM((CHUNK, 16), _f32),
        pltpu.VMEM_SHARED((NPAD, 16), _f32),
        pltpu.VMEM_SHARED((NPAD, 16), _f32),
    ],
)
def _deg_kernel(r0b, r1b, s0b, s1b, e1, zc,
                cru, cri, css, csd,
                idxg, e1v, acc0, acc1):
    cid = lax.axis_index("c")
    sid = lax.axis_index("s")
    rows = pl.ds(sid * RPT, RPT)
    pltpu.sync_copy(e1, e1v)
    pltpu.sync_copy(zc, acc0.at[rows])
    pltpu.sync_copy(zc, acc1.at[rows])
    plsc.subcore_barrier()

    def count(idx_hbm, acc, ngroups):
        @pl.loop(0, ngroups)
        def _(g):
            pltpu.sync_copy(idx_hbm.at[sid].at[pl.ds(g * G, G)], idxg)
            for k in range(G):
                pltpu.sync_copy(e1v, acc.at[idxg.at[k]], add=True)

    @pl.when(cid == 0)
    def _():
        count(r0b, acc0, GR)
        count(r1b, acc1, GR)

    @pl.when(cid == 1)
    def _():
        count(s0b, acc0, GS)
        count(s1b, acc1, GS)

    plsc.subcore_barrier()

    @pl.when(cid == 0)
    def _():
        pltpu.sync_copy(acc0.at[rows], cru.at[rows])
        pltpu.sync_copy(acc1.at[rows], cri.at[rows])

    @pl.when(cid == 1)
    def _():
        pltpu.sync_copy(acc0.at[rows], css.at[rows])
        pltpu.sync_copy(acc1.at[rows], csd.at[rows])


@functools.partial(
    pl.kernel,
    out_type=tuple(jax.ShapeDtypeStruct((NPAD, D), _f32) for _ in range(3)),
    mesh=_mesh,
    compiler_params=pltpu.CompilerParams(use_tc_tiling_on_sc=False),
    scratch_types=[
        pltpu.VMEM((G, CHUNK), _i32),
        pltpu.VMEM((G, CHUNK), _i32),
        pltpu.VMEM((CHUNK, D), _f32),
        pltpu.VMEM_SHARED((NPAD, D), _f32),
    ],
)
def _layer_kernel(hu, hi, gu, r0b, r1b, s0b, s1b, zh,
                  au_o, ai_o, bu_o,
                  srcg, dstg, rowbuf, acc):
    cid = lax.axis_index("c")
    sid = lax.axis_index("s")
    rows = pl.ds(sid * RPT, RPT)

    pltpu.sync_copy(zh, acc.at[rows])
    plsc.subcore_barrier()

    def scatter_pass(src_hbm, dst_hbm, tab, ngroups):
        @pl.loop(0, ngroups)
        def _(g):
            pltpu.sync_copy(src_hbm.at[sid].at[pl.ds(g * G, G)], srcg)
            pltpu.sync_copy(dst_hbm.at[sid].at[pl.ds(g * G, G)], dstg)
            for k in range(G):
                pltpu.sync_copy(tab.at[srcg.at[k]], rowbuf)
                pltpu.sync_copy(rowbuf, acc.at[dstg.at[k]], add=True)

    @pl.when(cid == 0)
    def _():
        scatter_pass(r1b, r0b, hi, GR)
        plsc.subcore_barrier()
        pltpu.sync_copy(acc.at[rows], au_o.at[rows])
        pltpu.sync_copy(zh, acc.at[rows])
        plsc.subcore_barrier()
        scatter_pass(s0b, s1b, gu, GS)
        plsc.subcore_barrier()
        pltpu.sync_copy(acc.at[rows], bu_o.at[rows])

    @pl.when(cid == 1)
    def _():
        scatter_pass(r0b, r1b, hu, GR)
        plsc.subcore_barrier()
        pltpu.sync_copy(acc.at[rows], ai_o.at[rows])


@functools.partial(
    pl.kernel,
    out_type=tuple(jax.ShapeDtypeStruct((32 * SCORE_PT,), _f32) for _ in range(2)),
    mesh=_mesh,
    compiler_params=pltpu.CompilerParams(use_tc_tiling_on_sc=False,
                                         needs_layout_passes=False),
    scratch_types=[
        pltpu.VMEM((CP, CHUNK), _i32),
        pltpu.VMEM((CP, CHUNK), _i32),
        pltpu.VMEM((CHUNK, D), _f32),
        pltpu.VMEM((CHUNK, D), _f32),
        pltpu.VMEM((SCORE_PT,), _f32),
    ],
)
def _score_kernel(ru, ri, pub, pib, nub, nib,
                  pos_o, neg_o,
                  ublk, iblk, urows, irows, sv):
    cid = lax.axis_index("c")
    sid = lax.axis_index("s")
    wid = sid * 2 + cid

    def one(ub_h, ib_h, out_h):
        pltpu.sync_copy(ub_h.at[wid], ublk)
        pltpu.sync_copy(ib_h.at[wid], iblk)

        @pl.loop(0, CP)
        def _(j):
            pltpu.sync_copy(ru.at[ublk.at[j]], urows)
            pltpu.sync_copy(ri.at[iblk.at[j]], irows)

            @pl.loop(0, CHUNK // 16)
            def _(g):
                eloc = g * 16 + lax.iota(_i32, 16)
                acc = jnp.zeros((16,), _f32)
                for dd in range(D):
                    col = jnp.full((16,), dd, _i32)
                    acc = acc + (plsc.load_gather(urows, [eloc, col])
                                 * plsc.load_gather(irows, [eloc, col]))
                sv[pl.ds(j * CHUNK + g * 16, 16)] = acc

        pltpu.sync_copy(sv, out_h.at[pl.ds(wid * SCORE_PT, SCORE_PT)])

    one(pub, pib, pos_o)
    one(nub, nib, neg_o)


def _edge_blocks(idx, per_tile, fill):
    idx = idx.astype(_i32)
    total = 16 * per_tile
    idx = jnp.pad(idx, (0, total - idx.shape[0]), constant_values=fill)
    return idx.reshape(16, per_tile // CHUNK, CHUNK)


def _score_blocks(idx):
    idx = idx.astype(_i32)
    idx = jnp.pad(idx, (0, 32 * SCORE_PT - idx.shape[0]))
    return idx.reshape(32, CP, CHUNK)


def kernel(user_emb, item_emb, edge_index_rate, edge_index_social,
           pos_edge_index, neg_edge_index):
    r0b = _edge_blocks(edge_index_rate[0], RATE_PT, TRASH)
    r1b = _edge_blocks(edge_index_rate[1], RATE_PT, TRASH)
    s0b = _edge_blocks(edge_index_social[0], SOC_PT, TRASH)
    s1b = _edge_blocks(edge_index_social[1], SOC_PT, TRASH)

    e1 = jnp.zeros((CHUNK, 16), _f32).at[:, 0].set(1.0)
    zc = jnp.zeros((RPT, 16), _f32)
    zh = jnp.zeros((RPT, D), _f32)

    cru, cri, css, csd = _deg_kernel(r0b, r1b, s0b, s1b, e1, zc)

    def scale(cnt):
        s = lax.rsqrt(jnp.clip(cnt[:N, 0], 1.0, None))
        return jnp.pad(s, (0, NPAD - N), constant_values=1.0)[:, None]

    au, bi, cs, ed = scale(cru), scale(cri), scale(css), scale(csd)

    cur_u = jnp.pad(user_emb, ((0, NPAD - N), (0, 0)))
    cur_i = jnp.pad(item_emb, ((0, NPAD - N), (0, 0)))
    res_u, res_i = cur_u, cur_i
    for _ in range(3):
        hu = au * cur_u
        hi = bi * cur_i
        gu = cs * cur_u
        Au, Ai, Bu = _layer_kernel(hu, hi, gu, r0b, r1b, s0b, s1b, zh)
        emb_u = au * Au + ed * Bu
        emb_i = bi * Ai
        res_u = res_u + emb_u
        res_i = res_i + emb_i
        cur_u, cur_i = emb_u, emb_i

    ru = res_u * 0.25
    ri = res_i * 0.25

    pub = _score_blocks(pos_edge_index[0])
    pib = _score_blocks(pos_edge_index[1])
    nub = _score_blocks(neg_edge_index[0])
    nib = _score_blocks(neg_edge_index[1])
    pos_s, neg_s = _score_kernel(ru, ri, pub, pib, nub, nib)
    return (pos_s[:P, None], neg_s[:P, None])


# speedup vs baseline: 7.1970x; 1.2231x over previous
import functools

import jax
import jax.numpy as jnp
from jax import lax
from jax.experimental import pallas as pl
from jax.experimental.pallas import tpu as pltpu
from jax.experimental.pallas import tpu_sc as plsc

N = 25000
D = 64
NPAD = 25088
RPT = NPAD // 16
TRASH = NPAD - 1

P = 100000

CHUNK = 128
G = 14
RATE_PT = 50176
CR = RATE_PT // CHUNK
GR = CR // G
SOC_PT = 25088
CS = SOC_PT // CHUNK
GS = CS // G
SCORE_PT = 3200
CP = SCORE_PT // CHUNK

_mesh = plsc.VectorSubcoreMesh(
    core_axis_name="c", subcore_axis_name="s", num_cores=2, num_subcores=16)

_f32 = jnp.float32
_i32 = jnp.int32


@functools.partial(
    pl.kernel,
    out_type=tuple(jax.ShapeDtypeStruct((NPAD, 16), _f32) for _ in range(4)),
    mesh=_mesh,
    compiler_params=pltpu.CompilerParams(use_tc_tiling_on_sc=False),
    scratch_types=[
        pltpu.VMEM((G, 2, CHUNK), _i32),
        pltpu.VMEM((CHUNK, 16), _f32),
        pltpu.VMEM_SHARED((NPAD, 16), _f32),
        pltpu.VMEM_SHARED((NPAD, 16), _f32),
        pltpu.SemaphoreType.DMA,
    ],
)
def _deg_kernel(rcb, scb, e1, zc,
                cru, cri, css, csd,
                idxg, e1v, acc0, acc1, dsem):
    cid = lax.axis_index("c")
    sid = lax.axis_index("s")
    rows = pl.ds(sid * RPT, RPT)
    pltpu.sync_copy(e1, e1v)
    pltpu.sync_copy(zc, acc0.at[rows])
    pltpu.sync_copy(zc, acc1.at[rows])
    plsc.subcore_barrier()

    def count(idx_hbm, ngroups):
        @pl.loop(0, ngroups)
        def _(g):
            pltpu.sync_copy(idx_hbm.at[sid].at[pl.ds(g * G, G)], idxg)

            @pl.loop(0, G)
            def _(k):
                pltpu.async_copy(e1v, acc0.at[idxg.at[k, 0]], dsem, add=True)
                pltpu.async_copy(e1v, acc1.at[idxg.at[k, 1]], dsem, add=True)

            @pl.loop(0, 2 * G)
            def _(k):
                pltpu.make_async_copy(e1v, acc0.at[idxg.at[0, 0]], dsem).wait()

    @pl.when(cid == 0)
    def _():
        count(rcb, GR)

    @pl.when(cid == 1)
    def _():
        count(scb, GS)

    plsc.subcore_barrier()

    @pl.when(cid == 0)
    def _():
        pltpu.sync_copy(acc0.at[rows], cru.at[rows])
        pltpu.sync_copy(acc1.at[rows], cri.at[rows])

    @pl.when(cid == 1)
    def _():
        pltpu.sync_copy(acc0.at[rows], css.at[rows])
        pltpu.sync_copy(acc1.at[rows], csd.at[rows])


@functools.partial(
    pl.kernel,
    out_type=tuple(jax.ShapeDtypeStruct((NPAD, D), _f32) for _ in range(3)),
    mesh=_mesh,
    compiler_params=pltpu.CompilerParams(use_tc_tiling_on_sc=False),
    scratch_types=[
        pltpu.VMEM((G, 2, CHUNK), _i32),
        pltpu.VMEM((G, 2, CHUNK), _i32),
        pltpu.VMEM((CHUNK, D), _f32),
        pltpu.VMEM((CHUNK, D), _f32),
        pltpu.VMEM_SHARED((NPAD, D), _f32),
        pltpu.SemaphoreType.DMA,
        pltpu.SemaphoreType.DMA,
        pltpu.SemaphoreType.DMA,
        pltpu.SemaphoreType.DMA,
    ],
)
def _layer_kernel(hu, hi, gu, rcb, scb, zh,
                  au_o, ai_o, bu_o,
                  ia, ib, rb0, rb1, acc, isem0, isem1, gsem0, gsem1):
    cid = lax.axis_index("c")
    sid = lax.axis_index("s")
    rows = pl.ds(sid * RPT, RPT)

    pltpu.sync_copy(zh, acc.at[rows])
    plsc.subcore_barrier()

    def scatter_pass(idx_hbm, srcsel, tab, ngroups):
        dstsel = 1 - srcsel
        ih = idx_hbm.at[sid]

        def process(ixb):
            pltpu.async_copy(tab.at[ixb.at[0, srcsel]], rb0, gsem0)

            @pl.loop(0, G // 2)
            def _(q):
                a = 2 * q
                nxt = jnp.minimum(a + 2, G - 1)
                pltpu.make_async_copy(tab.at[ixb.at[a, srcsel]], rb0,
                                      gsem0).wait()
                pltpu.async_copy(tab.at[ixb.at[a + 1, srcsel]], rb1, gsem1)
                pltpu.sync_copy(rb0, acc.at[ixb.at[a, dstsel]], add=True)
                pltpu.make_async_copy(tab.at[ixb.at[a + 1, srcsel]], rb1,
                                      gsem1).wait()
                pltpu.async_copy(tab.at[ixb.at[nxt, srcsel]], rb0, gsem0)
                pltpu.sync_copy(rb1, acc.at[ixb.at[a + 1, dstsel]], add=True)

            pltpu.make_async_copy(tab.at[ixb.at[G - 1, srcsel]], rb0,
                                  gsem0).wait()

        pltpu.sync_copy(ih.at[pl.ds(0, G)], ia)
        pltpu.async_copy(ih.at[pl.ds(G, G)], ib, isem1)

        @pl.loop(0, ngroups // 2)
        def _(t):
            process(ia)
            g2 = jnp.minimum(2 * t + 2, ngroups - 1)
            pltpu.async_copy(ih.at[pl.ds(g2 * G, G)], ia, isem0)
            pltpu.make_async_copy(ih.at[pl.ds(0, G)], ib, isem1).wait()
            process(ib)
            g3 = jnp.minimum(2 * t + 3, ngroups - 1)
            pltpu.async_copy(ih.at[pl.ds(g3 * G, G)], ib, isem1)
            pltpu.make_async_copy(ih.at[pl.ds(0, G)], ia, isem0).wait()

        pltpu.make_async_copy(ih.at[pl.ds(0, G)], ib, isem1).wait()

    @pl.when(cid == 0)
    def _():
        scatter_pass(rcb, 1, hi, GR)
        plsc.subcore_barrier()
        pltpu.sync_copy(acc.at[rows], au_o.at[rows])
        pltpu.sync_copy(zh, acc.at[rows])
        plsc.subcore_barrier()
        scatter_pass(scb, 0, gu, GS)
        plsc.subcore_barrier()
        pltpu.sync_copy(acc.at[rows], bu_o.at[rows])

    @pl.when(cid == 1)
    def _():
        scatter_pass(rcb, 0, hu, GR)
        plsc.subcore_barrier()
        pltpu.sync_copy(acc.at[rows], ai_o.at[rows])


@functools.partial(
    pl.kernel,
    out_type=tuple(jax.ShapeDtypeStruct((32 * SCORE_PT, D), _f32)
                   for _ in range(4)),
    mesh=_mesh,
    compiler_params=pltpu.CompilerParams(use_tc_tiling_on_sc=False),
    scratch_types=[
        pltpu.VMEM((CP, CHUNK), _i32),
        pltpu.VMEM((CHUNK, D), _f32),
        pltpu.VMEM((CHUNK, D), _f32),
        pltpu.SemaphoreType.DMA,
        pltpu.SemaphoreType.DMA,
    ],
)
def _score_gather_kernel(ru, ri, pub, pib, nub, nib,
                         ugp, igp, ugn, ign,
                         blk, rb0, rb1, gsem0, gsem1):
    cid = lax.axis_index("c")
    sid = lax.axis_index("s")
    wid = sid * 2 + cid
    base = wid * SCORE_PT

    def one(blk_h, tab, out_h):
        pltpu.sync_copy(blk_h.at[wid], blk)
        pltpu.async_copy(tab.at[blk.at[0]], rb0, gsem0)

        @pl.loop(0, CP // 2)
        def _(t):
            a = 2 * t
            pltpu.make_async_copy(tab.at[blk.at[a]], rb0, gsem0).wait()
            pltpu.async_copy(tab.at[blk.at[a + 1]], rb1, gsem1)
            pltpu.sync_copy(rb0, out_h.at[pl.ds(base + a * CHUNK, CHUNK)])
            pltpu.make_async_copy(tab.at[blk.at[a + 1]], rb1, gsem1).wait()
            nxt = jnp.minimum(a + 2, CP - 1)
            pltpu.async_copy(tab.at[blk.at[nxt]], rb0, gsem0)
            pltpu.sync_copy(rb1, out_h.at[pl.ds(base + (a + 1) * CHUNK, CHUNK)])

        pltpu.make_async_copy(tab.at[blk.at[CP - 1]], rb0, gsem0).wait()
        pltpu.sync_copy(rb0, out_h.at[pl.ds(base + (CP - 1) * CHUNK, CHUNK)])

    one(pub, ru, ugp)
    one(pib, ri, igp)
    one(nub, ru, ugn)
    one(nib, ri, ign)


_DOT_BLOCK = 1024


def _dot_body(up, ip, un, inn, po, no):
    po[...] = jnp.sum(up[...] * ip[...], axis=1, keepdims=True)
    no[...] = jnp.sum(un[...] * inn[...], axis=1, keepdims=True)


_dot_kernel = pl.pallas_call(
    _dot_body,
    grid=(32 * SCORE_PT // _DOT_BLOCK,),
    in_specs=[pl.BlockSpec((_DOT_BLOCK, D), lambda b: (b, 0))] * 4,
    out_specs=[pl.BlockSpec((_DOT_BLOCK, 1), lambda b: (b, 0))] * 2,
    out_shape=[jax.ShapeDtypeStruct((32 * SCORE_PT, 1), _f32)] * 2,
)


def _edge_blocks(idx, per_tile, fill):
    idx = idx.astype(_i32)
    total = 16 * per_tile
    idx = jnp.pad(idx, (0, total - idx.shape[0]), constant_values=fill)
    return idx.reshape(16, per_tile // CHUNK, CHUNK)


def _score_blocks(idx):
    idx = idx.astype(_i32)
    idx = jnp.pad(idx, (0, 32 * SCORE_PT - idx.shape[0]))
    return idx.reshape(32, CP, CHUNK)


def kernel(user_emb, item_emb, edge_index_rate, edge_index_social,
           pos_edge_index, neg_edge_index):
    rcb = jnp.stack([_edge_blocks(edge_index_rate[0], RATE_PT, TRASH),
                     _edge_blocks(edge_index_rate[1], RATE_PT, TRASH)], axis=2)
    scb = jnp.stack([_edge_blocks(edge_index_social[0], SOC_PT, TRASH),
                     _edge_blocks(edge_index_social[1], SOC_PT, TRASH)], axis=2)

    e1 = jnp.zeros((CHUNK, 16), _f32).at[:, 0].set(1.0)
    zc = jnp.zeros((RPT, 16), _f32)
    zh = jnp.zeros((RPT, D), _f32)

    cru, cri, css, csd = _deg_kernel(rcb, scb, e1, zc)

    def scale(cnt):
        s = lax.rsqrt(jnp.clip(cnt[:N, 0], 1.0, None))
        return jnp.pad(s, (0, NPAD - N), constant_values=1.0)[:, None]

    au, bi, cs, ed = scale(cru), scale(cri), scale(css), scale(csd)

    cur_u = jnp.pad(user_emb, ((0, NPAD - N), (0, 0)))
    cur_i = jnp.pad(item_emb, ((0, NPAD - N), (0, 0)))
    res_u, res_i = cur_u, cur_i
    for _ in range(3):
        hu = au * cur_u
        hi = bi * cur_i
        gu = cs * cur_u
        Au, Ai, Bu = _layer_kernel(hu, hi, gu, rcb, scb, zh)
        emb_u = au * Au + ed * Bu
        emb_i = bi * Ai
        res_u = res_u + emb_u
        res_i = res_i + emb_i
        cur_u, cur_i = emb_u, emb_i

    ru = res_u * 0.25
    ri = res_i * 0.25

    pub = _score_blocks(pos_edge_index[0])
    pib = _score_blocks(pos_edge_index[1])
    nub = _score_blocks(neg_edge_index[0])
    nib = _score_blocks(neg_edge_index[1])
    ugp, igp, ugn, ign = _score_gather_kernel(ru, ri, pub, pib, nub, nib)
    pos_s, neg_s = _dot_kernel(ugp, igp, ugn, ign)
    return (pos_s[:P], neg_s[:P])


# speedup vs baseline: 7.4642x; 1.0371x over previous
import functools

import jax
import jax.numpy as jnp
from jax import lax
from jax.experimental import pallas as pl
from jax.experimental.pallas import tpu as pltpu
from jax.experimental.pallas import tpu_sc as plsc

N = 25000
D = 64
NPAD = 25088
RPT = NPAD // 16
TRASH = NPAD - 1

P = 100000

CHUNK = 128
G = 7
PAIR = 2 * G
RATE_PT = 50176
CR = RATE_PT // CHUNK
GR = CR // G
SOC_PT = 25088
CS = SOC_PT // CHUNK
CSH = CS // 2
GSH = CSH // G
SCORE_PT = 3456
CP = SCORE_PT // CHUNK

_mesh = plsc.VectorSubcoreMesh(
    core_axis_name="c", subcore_axis_name="s", num_cores=2, num_subcores=16)

_f32 = jnp.float32
_i32 = jnp.int32


@functools.partial(
    pl.kernel,
    out_type=tuple(jax.ShapeDtypeStruct((NPAD, 16), _f32) for _ in range(4)),
    mesh=_mesh,
    compiler_params=pltpu.CompilerParams(use_tc_tiling_on_sc=False),
    scratch_types=[
        pltpu.VMEM((G, 2, CHUNK), _i32),
        pltpu.VMEM((CHUNK, 16), _f32),
        pltpu.VMEM_SHARED((NPAD, 16), _f32),
        pltpu.VMEM_SHARED((NPAD, 16), _f32),
        pltpu.SemaphoreType.DMA,
    ],
)
def _deg_kernel(rcb, scb, e1, zc,
                cru, cri, css, csd,
                idxg, e1v, acc0, acc1, dsem):
    cid = lax.axis_index("c")
    sid = lax.axis_index("s")
    rows = pl.ds(sid * RPT, RPT)
    pltpu.sync_copy(e1, e1v)
    pltpu.sync_copy(zc, acc0.at[rows])
    pltpu.sync_copy(zc, acc1.at[rows])
    plsc.subcore_barrier()

    def count(idx_hbm, ngroups):
        @pl.loop(0, ngroups)
        def _(g):
            pltpu.sync_copy(idx_hbm.at[sid].at[pl.ds(g * G, G)], idxg)

            @pl.loop(0, G)
            def _(k):
                pltpu.async_copy(e1v, acc0.at[idxg.at[k, 0]], dsem, add=True)
                pltpu.async_copy(e1v, acc1.at[idxg.at[k, 1]], dsem, add=True)

            @pl.loop(0, 2 * G)
            def _(k):
                pltpu.make_async_copy(e1v, acc0.at[idxg.at[0, 0]], dsem).wait()

    @pl.when(cid == 0)
    def _():
        count(rcb, GR)

    @pl.when(cid == 1)
    def _():
        count(scb, GR // 2)

    plsc.subcore_barrier()

    @pl.when(cid == 0)
    def _():
        pltpu.sync_copy(acc0.at[rows], cru.at[rows])
        pltpu.sync_copy(acc1.at[rows], cri.at[rows])

    @pl.when(cid == 1)
    def _():
        pltpu.sync_copy(acc0.at[rows], css.at[rows])
        pltpu.sync_copy(acc1.at[rows], csd.at[rows])


@functools.partial(
    pl.kernel,
    out_type=tuple(jax.ShapeDtypeStruct((NPAD, D), _f32) for _ in range(4)),
    mesh=_mesh,
    compiler_params=pltpu.CompilerParams(use_tc_tiling_on_sc=False),
    scratch_types=[
        pltpu.VMEM((G, 2, CHUNK), _i32),
        pltpu.VMEM((G, 2, CHUNK), _i32),
        pltpu.VMEM((CHUNK, D), _f32),
        pltpu.VMEM((CHUNK, D), _f32),
        pltpu.VMEM((CHUNK, D), _f32),
        pltpu.VMEM_SHARED((NPAD, D), _f32),
        pltpu.SemaphoreType.DMA,
        pltpu.SemaphoreType.DMA,
        pltpu.SemaphoreType.DMA,
        pltpu.SemaphoreType.DMA,
        pltpu.SemaphoreType.DMA,
    ],
)
def _layer_kernel(hu, hi, gu, rcb, scb0, scb1, zh,
                  au_o, ai_o, bu0_o, bu1_o,
                  ia, ib, rb0, rb1, rb2, acc,
                  isem0, isem1, gsem0, gsem1, gsem2):
    cid = lax.axis_index("c")
    sid = lax.axis_index("s")
    rows = pl.ds(sid * RPT, RPT)
    rb = (rb0, rb1, rb2)
    gs = (gsem0, gsem1, gsem2)

    pltpu.sync_copy(zh, acc.at[rows])
    plsc.subcore_barrier()

    def scatter_pass(idx_hbm, srcsel, tab, ngroups):
        dstsel = 1 - srcsel
        ih = idx_hbm.at[sid]

        def idxrow(c, which):
            buf = ia if c < G else ib
            return buf.at[c % G, which]

        pltpu.sync_copy(ih.at[pl.ds(0, G)], ia)
        pltpu.async_copy(ih.at[pl.ds(G, G)], ib, isem1)
        pltpu.async_copy(tab.at[ia.at[0, srcsel]], rb0, gsem0)
        pltpu.async_copy(tab.at[ia.at[1, srcsel]], rb1, gsem1)

        @pl.loop(0, ngroups // 2)
        def _(t):
            for c in range(PAIR):
                if c == G - 2:
                    pltpu.make_async_copy(ih.at[pl.ds(0, G)], ib, isem1).wait()
                b = c % 3
                pltpu.make_async_copy(tab.at[idxrow(c, srcsel)], rb[b],
                                      gs[b]).wait()
                if c + 2 < PAIR:
                    nb = (c + 2) % 3
                    pltpu.async_copy(tab.at[idxrow(c + 2, srcsel)], rb[nb],
                                     gs[nb])
                pltpu.sync_copy(rb[b], acc.at[idxrow(c, dstsel)], add=True)
                if c == G - 1:
                    g2 = jnp.minimum(2 * t + 2, ngroups - 1)
                    pltpu.async_copy(ih.at[pl.ds(g2 * G, G)], ia, isem0)
            g3 = jnp.minimum(2 * t + 3, ngroups - 1)
            pltpu.async_copy(ih.at[pl.ds(g3 * G, G)], ib, isem1)
            pltpu.make_async_copy(ih.at[pl.ds(0, G)], ia, isem0).wait()
            pltpu.async_copy(tab.at[ia.at[0, srcsel]], rb0, gsem0)
            pltpu.async_copy(tab.at[ia.at[1, srcsel]], rb1, gsem1)

        pltpu.make_async_copy(tab.at[ia.at[0, srcsel]], rb0, gsem0).wait()
        pltpu.make_async_copy(tab.at[ia.at[1, srcsel]], rb1, gsem1).wait()
        pltpu.make_async_copy(ih.at[pl.ds(0, G)], ib, isem1).wait()

    def core_work(rate_srcsel, rate_tab, rate_out, soc_blk, soc_out):
        scatter_pass(rcb, rate_srcsel, rate_tab, GR)
        plsc.subcore_barrier()
        pltpu.sync_copy(acc.at[rows], rate_out.at[rows])
        pltpu.sync_copy(zh, acc.at[rows])
        plsc.subcore_barrier()
        scatter_pass(soc_blk, 0, gu, GSH)
        plsc.subcore_barrier()
        pltpu.sync_copy(acc.at[rows], soc_out.at[rows])

    @pl.when(cid == 0)
    def _():
        core_work(1, hi, au_o, scb0, bu0_o)

    @pl.when(cid == 1)
    def _():
        core_work(0, hu, ai_o, scb1, bu1_o)


@functools.partial(
    pl.kernel,
    out_type=tuple(jax.ShapeDtypeStruct((32 * SCORE_PT, D), _f32)
                   for _ in range(4)),
    mesh=_mesh,
    compiler_params=pltpu.CompilerParams(use_tc_tiling_on_sc=False),
    scratch_types=[
        pltpu.VMEM((CP, CHUNK), _i32),
        pltpu.VMEM((CHUNK, D), _f32),
        pltpu.VMEM((CHUNK, D), _f32),
        pltpu.VMEM((CHUNK, D), _f32),
        pltpu.SemaphoreType.DMA,
        pltpu.SemaphoreType.DMA,
        pltpu.SemaphoreType.DMA,
    ],
)
def _score_gather_kernel(ru, ri, pub, pib, nub, nib,
                         ugp, igp, ugn, ign,
                         blk, rb0, rb1, rb2, gsem0, gsem1, gsem2):
    cid = lax.axis_index("c")
    sid = lax.axis_index("s")
    wid = sid * 2 + cid
    base = wid * SCORE_PT
    rb = (rb0, rb1, rb2)
    gs = (gsem0, gsem1, gsem2)

    def one(blk_h, tab, out_h):
        pltpu.sync_copy(blk_h.at[wid], blk)
        pltpu.async_copy(tab.at[blk.at[0]], rb0, gsem0)
        pltpu.async_copy(tab.at[blk.at[1]], rb1, gsem1)

        @pl.loop(0, CP // 3)
        def _(t):
            for o in range(3):
                c = 3 * t + o
                pltpu.make_async_copy(tab.at[blk.at[c]], rb[o], gs[o]).wait()
                nxt = jnp.minimum(c + 2, CP - 1)
                nb = (o + 2) % 3
                pltpu.async_copy(tab.at[blk.at[nxt]], rb[nb], gs[nb])
                pltpu.sync_copy(rb[o],
                                out_h.at[pl.ds(base + c * CHUNK, CHUNK)])

        pltpu.make_async_copy(tab.at[blk.at[CP - 1]], rb0, gsem0).wait()
        pltpu.make_async_copy(tab.at[blk.at[CP - 1]], rb1, gsem1).wait()

    one(pub, ru, ugp)
    one(pib, ri, igp)
    one(nub, ru, ugn)
    one(nib, ri, ign)


_DOT_BLOCK = 1024


def _dot_body(up, ip, un, inn, po, no):
    po[...] = jnp.sum(up[...] * ip[...], axis=1, keepdims=True)
    no[...] = jnp.sum(un[...] * inn[...], axis=1, keepdims=True)


_dot_kernel = pl.pallas_call(
    _dot_body,
    grid=(32 * SCORE_PT // _DOT_BLOCK,),
    in_specs=[pl.BlockSpec((_DOT_BLOCK, D), lambda b: (b, 0))] * 4,
    out_specs=[pl.BlockSpec((_DOT_BLOCK, 1), lambda b: (b, 0))] * 2,
    out_shape=[jax.ShapeDtypeStruct((32 * SCORE_PT, 1), _f32)] * 2,
)


def _edge_blocks(idx, per_tile, fill):
    idx = idx.astype(_i32)
    total = 16 * per_tile
    idx = jnp.pad(idx, (0, total - idx.shape[0]), constant_values=fill)
    return idx.reshape(16, per_tile // CHUNK, CHUNK)


def _score_blocks(idx):
    idx = idx.astype(_i32)
    idx = jnp.pad(idx, (0, 32 * SCORE_PT - idx.shape[0]))
    return idx.reshape(32, CP, CHUNK)


def kernel(user_emb, item_emb, edge_index_rate, edge_index_social,
           pos_edge_index, neg_edge_index):
    rcb = jnp.stack([_edge_blocks(edge_index_rate[0], RATE_PT, TRASH),
                     _edge_blocks(edge_index_rate[1], RATE_PT, TRASH)], axis=2)
    scb = jnp.stack([_edge_blocks(edge_index_social[0], SOC_PT, TRASH),
                     _edge_blocks(edge_index_social[1], SOC_PT, TRASH)], axis=2)
    scb0 = scb[:, :CSH]
    scb1 = scb[:, CSH:]

    e1 = jnp.zeros((CHUNK, 16), _f32).at[:, 0].set(1.0)
    zc = jnp.zeros((RPT, 16), _f32)
    zh = jnp.zeros((RPT, D), _f32)

    cru, cri, css, csd = _deg_kernel(rcb, scb, e1, zc)

    def scale(cnt):
        s = lax.rsqrt(jnp.clip(cnt[:N, 0], 1.0, None))
        return jnp.pad(s, (0, NPAD - N), constant_values=1.0)[:, None]

    au, bi, cs, ed = scale(cru), scale(cri), scale(css), scale(csd)

    cur_u = jnp.pad(user_emb, ((0, NPAD - N), (0, 0)))
    cur_i = jnp.pad(item_emb, ((0, NPAD - N), (0, 0)))
    res_u, res_i = cur_u, cur_i
    for _ in range(3):
        hu = au * cur_u
        hi = bi * cur_i
        gu = cs * cur_u
        Au, Ai, Bu0, Bu1 = _layer_kernel(hu, hi, gu, rcb, scb0, scb1, zh)
        emb_u = au * Au + ed * (Bu0 + Bu1)
        emb_i = bi * Ai
        res_u = res_u + emb_u
        res_i = res_i + emb_i
        cur_u, cur_i = emb_u, emb_i

    ru = res_u * 0.25
    ri = res_i * 0.25

    pub = _score_blocks(pos_edge_index[0])
    pib = _score_blocks(pos_edge_index[1])
    nub = _score_blocks(neg_edge_index[0])
    nib = _score_blocks(neg_edge_index[1])
    ugp, igp, ugn, ign = _score_gather_kernel(ru, ri, pub, pib, nub, nib)
    pos_s, neg_s = _dot_kernel(ugp, igp, ugn, ign)
    return (pos_s[:P], neg_s[:P])


# speedup vs baseline: 7.5805x; 1.0156x over previous
import functools

import jax
import jax.numpy as jnp
from jax import lax
from jax.experimental import pallas as pl
from jax.experimental.pallas import tpu as pltpu
from jax.experimental.pallas import tpu_sc as plsc

N = 25000
D = 64
NPAD = 25088
RPT = NPAD // 16
TRASH = NPAD - 1

P = 100000

CHUNK = 128
G = 7
PAIR = 2 * G
RATE_PT = 50176
CR = RATE_PT // CHUNK
GR = CR // G
SOC_PT = 25088
CS = SOC_PT // CHUNK
CSH = CS // 2
GSH = CSH // G
SCORE_PT = 3456
CP = SCORE_PT // CHUNK

_mesh = plsc.VectorSubcoreMesh(
    core_axis_name="c", subcore_axis_name="s", num_cores=2, num_subcores=16)

_f32 = jnp.float32
_i32 = jnp.int32


@functools.partial(
    pl.kernel,
    out_type=tuple(jax.ShapeDtypeStruct((NPAD, 16), _f32) for _ in range(4)),
    mesh=_mesh,
    compiler_params=pltpu.CompilerParams(use_tc_tiling_on_sc=False),
    scratch_types=[
        pltpu.VMEM((G, 2, CHUNK), _i32),
        pltpu.VMEM((CHUNK, 16), _f32),
        pltpu.VMEM_SHARED((NPAD, 16), _f32),
        pltpu.VMEM_SHARED((NPAD, 16), _f32),
        pltpu.SemaphoreType.DMA,
    ],
)
def _deg_kernel(rcb, scb, e1, zc,
                cru, cri, css, csd,
                idxg, e1v, acc0, acc1, dsem):
    cid = lax.axis_index("c")
    sid = lax.axis_index("s")
    rows = pl.ds(sid * RPT, RPT)
    pltpu.sync_copy(e1, e1v)
    pltpu.sync_copy(zc, acc0.at[rows])
    pltpu.sync_copy(zc, acc1.at[rows])
    plsc.subcore_barrier()

    def count(idx_hbm, ngroups):
        @pl.loop(0, ngroups)
        def _(g):
            pltpu.sync_copy(idx_hbm.at[sid].at[pl.ds(g * G, G)], idxg)

            @pl.loop(0, G)
            def _(k):
                pltpu.async_copy(e1v, acc0.at[idxg.at[k, 0]], dsem, add=True)
                pltpu.async_copy(e1v, acc1.at[idxg.at[k, 1]], dsem, add=True)

            @pl.loop(0, 2 * G)
            def _(k):
                pltpu.make_async_copy(e1v, acc0.at[idxg.at[0, 0]], dsem).wait()

    @pl.when(cid == 0)
    def _():
        count(rcb, GR)

    @pl.when(cid == 1)
    def _():
        count(scb, GR // 2)

    plsc.subcore_barrier()

    @pl.when(cid == 0)
    def _():
        pltpu.sync_copy(acc0.at[rows], cru.at[rows])
        pltpu.sync_copy(acc1.at[rows], cri.at[rows])

    @pl.when(cid == 1)
    def _():
        pltpu.sync_copy(acc0.at[rows], css.at[rows])
        pltpu.sync_copy(acc1.at[rows], csd.at[rows])


@functools.partial(
    pl.kernel,
    out_type=tuple(jax.ShapeDtypeStruct((NPAD, D), _f32) for _ in range(4)),
    mesh=_mesh,
    compiler_params=pltpu.CompilerParams(use_tc_tiling_on_sc=False),
    scratch_types=[
        pltpu.VMEM((G, 2, CHUNK), _i32),
        pltpu.VMEM((G, 2, CHUNK), _i32),
        pltpu.VMEM((CHUNK, D), _f32),
        pltpu.VMEM((CHUNK, D), _f32),
        pltpu.VMEM((CHUNK, D), _f32),
        pltpu.VMEM_SHARED((NPAD, D), _f32),
        pltpu.SemaphoreType.DMA,
        pltpu.SemaphoreType.DMA,
        pltpu.SemaphoreType.DMA,
        pltpu.SemaphoreType.DMA,
        pltpu.SemaphoreType.DMA,
    ],
)
def _layer_kernel(hu, hi, gu, rcb, scb0, scb1, zh,
                  au_o, ai_o, bu0_o, bu1_o,
                  ia, ib, rb0, rb1, rb2, acc,
                  isem0, isem1, gsem0, gsem1, gsem2):
    cid = lax.axis_index("c")
    sid = lax.axis_index("s")
    rows = pl.ds(sid * RPT, RPT)
    rb = (rb0, rb1, rb2)
    gs = (gsem0, gsem1, gsem2)

    pltpu.sync_copy(zh, acc.at[rows])
    plsc.subcore_barrier()

    def scatter_pass(idx_hbm, srcsel, tab, ngroups):
        dstsel = 1 - srcsel
        ih = idx_hbm.at[sid]

        def idxrow(c, which):
            buf = ia if c < G else ib
            return buf.at[c % G, which]

        pltpu.sync_copy(ih.at[pl.ds(0, G)], ia)
        pltpu.async_copy(ih.at[pl.ds(G, G)], ib, isem1)
        pltpu.async_copy(tab.at[ia.at[0, srcsel]], rb0, gsem0)
        pltpu.async_copy(tab.at[ia.at[1, srcsel]], rb1, gsem1)

        @pl.loop(0, ngroups // 2)
        def _(t):
            for c in range(PAIR):
                if c == G - 2:
                    pltpu.make_async_copy(ih.at[pl.ds(0, G)], ib, isem1).wait()
                b = c % 3
                pltpu.make_async_copy(tab.at[idxrow(c, srcsel)], rb[b],
                                      gs[b]).wait()
                if c + 2 < PAIR:
                    nb = (c + 2) % 3
                    pltpu.async_copy(tab.at[idxrow(c + 2, srcsel)], rb[nb],
                                     gs[nb])
                pltpu.sync_copy(rb[b], acc.at[idxrow(c, dstsel)], add=True)
                if c == G - 1:
                    g2 = jnp.minimum(2 * t + 2, ngroups - 1)
                    pltpu.async_copy(ih.at[pl.ds(g2 * G, G)], ia, isem0)
            g3 = jnp.minimum(2 * t + 3, ngroups - 1)
            pltpu.async_copy(ih.at[pl.ds(g3 * G, G)], ib, isem1)
            pltpu.make_async_copy(ih.at[pl.ds(0, G)], ia, isem0).wait()
            pltpu.async_copy(tab.at[ia.at[0, srcsel]], rb0, gsem0)
            pltpu.async_copy(tab.at[ia.at[1, srcsel]], rb1, gsem1)

        pltpu.make_async_copy(tab.at[ia.at[0, srcsel]], rb0, gsem0).wait()
        pltpu.make_async_copy(tab.at[ia.at[1, srcsel]], rb1, gsem1).wait()
        pltpu.make_async_copy(ih.at[pl.ds(0, G)], ib, isem1).wait()

    def core_work(rate_srcsel, rate_tab, rate_out, soc_blk, soc_out):
        scatter_pass(rcb, rate_srcsel, rate_tab, GR)
        plsc.subcore_barrier()
        pltpu.sync_copy(acc.at[rows], rate_out.at[rows])
        pltpu.sync_copy(zh, acc.at[rows])
        plsc.subcore_barrier()
        scatter_pass(soc_blk, 0, gu, GSH)
        plsc.subcore_barrier()
        pltpu.sync_copy(acc.at[rows], soc_out.at[rows])

    @pl.when(cid == 0)
    def _():
        core_work(1, hi, au_o, scb0, bu0_o)

    @pl.when(cid == 1)
    def _():
        core_work(0, hu, ai_o, scb1, bu1_o)


@functools.partial(
    pl.kernel,
    out_type=tuple(jax.ShapeDtypeStruct((32 * SCORE_PT, D), _f32)
                   for _ in range(4)),
    mesh=_mesh,
    compiler_params=pltpu.CompilerParams(use_tc_tiling_on_sc=False),
    scratch_types=[
        pltpu.VMEM((CP, CHUNK), _i32),
        pltpu.VMEM((CHUNK, D), _f32),
        pltpu.VMEM((CHUNK, D), _f32),
        pltpu.VMEM((CHUNK, D), _f32),
        pltpu.SemaphoreType.DMA,
        pltpu.SemaphoreType.DMA,
        pltpu.SemaphoreType.DMA,
    ],
)
def _score_gather_kernel(ru, ri, pub, pib, nub, nib,
                         ugp, igp, ugn, ign,
                         blk, rb0, rb1, rb2, gsem0, gsem1, gsem2):
    cid = lax.axis_index("c")
    sid = lax.axis_index("s")
    wid = sid * 2 + cid
    base = wid * SCORE_PT
    rb = (rb0, rb1, rb2)
    gs = (gsem0, gsem1, gsem2)

    def one(blk_h, tab, out_h):
        pltpu.sync_copy(blk_h.at[wid], blk)
        pltpu.async_copy(tab.at[blk.at[0]], rb0, gsem0)

        @pl.loop(0, CP // 2)
        def _(t):
            a = 2 * t
            pltpu.make_async_copy(tab.at[blk.at[a]], rb0, gsem0).wait()
            pltpu.async_copy(tab.at[blk.at[a + 1]], rb1, gsem1)
            pltpu.sync_copy(rb0, out_h.at[pl.ds(base + a * CHUNK, CHUNK)])
            pltpu.make_async_copy(tab.at[blk.at[a + 1]], rb1, gsem1).wait()
            nxt = jnp.minimum(a + 2, CP - 1)
            pltpu.async_copy(tab.at[blk.at[nxt]], rb0, gsem0)
            pltpu.sync_copy(rb1, out_h.at[pl.ds(base + (a + 1) * CHUNK, CHUNK)])

        pltpu.make_async_copy(tab.at[blk.at[CP - 1]], rb0, gsem0).wait()
        pltpu.sync_copy(rb0, out_h.at[pl.ds(base + (CP - 1) * CHUNK, CHUNK)])

    one(pub, ru, ugp)
    one(pib, ri, igp)
    one(nub, ru, ugn)
    one(nib, ri, ign)


_DOT_BLOCK = 1024


def _dot_body(up, ip, un, inn, po, no):
    po[...] = jnp.sum(up[...] * ip[...], axis=1, keepdims=True)
    no[...] = jnp.sum(un[...] * inn[...], axis=1, keepdims=True)


_dot_kernel = pl.pallas_call(
    _dot_body,
    grid=(32 * SCORE_PT // _DOT_BLOCK,),
    in_specs=[pl.BlockSpec((_DOT_BLOCK, D), lambda b: (b, 0))] * 4,
    out_specs=[pl.BlockSpec((_DOT_BLOCK, 1), lambda b: (b, 0))] * 2,
    out_shape=[jax.ShapeDtypeStruct((32 * SCORE_PT, 1), _f32)] * 2,
)


def _edge_blocks(idx, per_tile, fill):
    idx = idx.astype(_i32)
    total = 16 * per_tile
    idx = jnp.pad(idx, (0, total - idx.shape[0]), constant_values=fill)
    return idx.reshape(16, per_tile // CHUNK, CHUNK)


def _score_blocks(idx):
    idx = idx.astype(_i32)
    idx = jnp.pad(idx, (0, 32 * SCORE_PT - idx.shape[0]))
    return idx.reshape(32, CP, CHUNK)


def kernel(user_emb, item_emb, edge_index_rate, edge_index_social,
           pos_edge_index, neg_edge_index):
    rcb = jnp.stack([_edge_blocks(edge_index_rate[0], RATE_PT, TRASH),
                     _edge_blocks(edge_index_rate[1], RATE_PT, TRASH)], axis=2)
    scb = jnp.stack([_edge_blocks(edge_index_social[0], SOC_PT, TRASH),
                     _edge_blocks(edge_index_social[1], SOC_PT, TRASH)], axis=2)
    scb0 = scb[:, :CSH]
    scb1 = scb[:, CSH:]

    e1 = jnp.zeros((CHUNK, 16), _f32).at[:, 0].set(1.0)
    zc = jnp.zeros((RPT, 16), _f32)
    zh = jnp.zeros((RPT, D), _f32)

    cru, cri, css, csd = _deg_kernel(rcb, scb, e1, zc)

    def scale(cnt):
        s = lax.rsqrt(jnp.clip(cnt[:N, 0], 1.0, None))
        return jnp.pad(s, (0, NPAD - N), constant_values=1.0)[:, None]

    au, bi, cs, ed = scale(cru), scale(cri), scale(css), scale(csd)

    cur_u = jnp.pad(user_emb, ((0, NPAD - N), (0, 0)))
    cur_i = jnp.pad(item_emb, ((0, NPAD - N), (0, 0)))
    res_u, res_i = cur_u, cur_i
    for _ in range(3):
        hu = au * cur_u
        hi = bi * cur_i
        gu = cs * cur_u
        Au, Ai, Bu0, Bu1 = _layer_kernel(hu, hi, gu, rcb, scb0, scb1, zh)
        emb_u = au * Au + ed * (Bu0 + Bu1)
        emb_i = bi * Ai
        res_u = res_u + emb_u
        res_i = res_i + emb_i
        cur_u, cur_i = emb_u, emb_i

    ru = res_u * 0.25
    ri = res_i * 0.25

    pub = _score_blocks(pos_edge_index[0])
    pib = _score_blocks(pos_edge_index[1])
    nub = _score_blocks(neg_edge_index[0])
    nib = _score_blocks(neg_edge_index[1])
    ugp, igp, ugn, ign = _score_gather_kernel(ru, ri, pub, pib, nub, nib)
    pos_s, neg_s = _dot_kernel(ugp, igp, ugn, ign)
    return (pos_s[:P], neg_s[:P])


# speedup vs baseline: 8.8701x; 1.1701x over previous
import functools

import jax
import jax.numpy as jnp
from jax import lax
from jax.experimental import pallas as pl
from jax.experimental.pallas import tpu as pltpu
from jax.experimental.pallas import tpu_sc as plsc

N = 25000
D = 64
NPAD = 25088
RPT = NPAD // 16
TRASH = NPAD - 1

P = 100000

CHUNK = 128
G = 7
PAIR = 2 * G
RATE_PT = 50176
CR = RATE_PT // CHUNK
GR = CR // G
SOC_PT = 25088
CS = SOC_PT // CHUNK
CSH = CS // 2
GSH = CSH // G
SCORE_PT = 3456
CP = SCORE_PT // CHUNK

_mesh = plsc.VectorSubcoreMesh(
    core_axis_name="c", subcore_axis_name="s", num_cores=2, num_subcores=16)

_f32 = jnp.float32
_i32 = jnp.int32


@functools.partial(
    pl.kernel,
    out_type=tuple(jax.ShapeDtypeStruct((NPAD, 16), _f32) for _ in range(4)),
    mesh=_mesh,
    compiler_params=pltpu.CompilerParams(use_tc_tiling_on_sc=False),
    scratch_types=[
        pltpu.VMEM((G, 2, CHUNK), _i32),
        pltpu.VMEM((CHUNK, 16), _f32),
        pltpu.VMEM_SHARED((NPAD, 16), _f32),
        pltpu.VMEM_SHARED((NPAD, 16), _f32),
        pltpu.SemaphoreType.DMA,
    ],
)
def _deg_kernel(rcb, scb, e1, zc,
                cru, cri, css, csd,
                idxg, e1v, acc0, acc1, dsem):
    cid = lax.axis_index("c")
    sid = lax.axis_index("s")
    rows = pl.ds(sid * RPT, RPT)
    pltpu.sync_copy(e1, e1v)
    pltpu.sync_copy(zc, acc0.at[rows])
    pltpu.sync_copy(zc, acc1.at[rows])
    plsc.subcore_barrier()

    def count(idx_hbm, endsel, acc, ngroups):
        @pl.loop(0, ngroups)
        def _(g):
            pltpu.sync_copy(idx_hbm.at[sid].at[pl.ds(g * G, G)], idxg)

            @pl.loop(0, G)
            def _(k):
                pltpu.async_copy(e1v, acc.at[idxg.at[k, endsel]], dsem,
                                 add=True)

            @pl.loop(0, G)
            def _(k):
                pltpu.make_async_copy(e1v, acc.at[idxg.at[0, 0]], dsem).wait()

    @pl.when(cid == 0)
    def _():
        count(rcb, 0, acc0, GR)
        count(scb, 0, acc1, GR // 2)

    @pl.when(cid == 1)
    def _():
        count(rcb, 1, acc0, GR)
        count(scb, 1, acc1, GR // 2)

    plsc.subcore_barrier()

    @pl.when(cid == 0)
    def _():
        pltpu.sync_copy(acc0.at[rows], cru.at[rows])
        pltpu.sync_copy(acc1.at[rows], css.at[rows])

    @pl.when(cid == 1)
    def _():
        pltpu.sync_copy(acc0.at[rows], cri.at[rows])
        pltpu.sync_copy(acc1.at[rows], csd.at[rows])


@functools.partial(
    pl.kernel,
    out_type=tuple(jax.ShapeDtypeStruct((NPAD, D), _f32) for _ in range(4)),
    mesh=_mesh,
    compiler_params=pltpu.CompilerParams(use_tc_tiling_on_sc=False),
    scratch_types=[
        pltpu.VMEM((G, 2, CHUNK), _i32),
        pltpu.VMEM((G, 2, CHUNK), _i32),
        pltpu.VMEM((CHUNK, D), _f32),
        pltpu.VMEM((CHUNK, D), _f32),
        pltpu.VMEM((CHUNK, D), _f32),
        pltpu.VMEM_SHARED((NPAD, D), _f32),
        pltpu.SemaphoreType.DMA,
        pltpu.SemaphoreType.DMA,
        pltpu.SemaphoreType.DMA,
        pltpu.SemaphoreType.DMA,
        pltpu.SemaphoreType.DMA,
    ],
)
def _layer_kernel(hu, hi, gu, rcb, scb0, scb1, zh,
                  au_o, ai_o, bu0_o, bu1_o,
                  ia, ib, rb0, rb1, rb2, acc,
                  isem0, isem1, gsem0, gsem1, gsem2):
    cid = lax.axis_index("c")
    sid = lax.axis_index("s")
    rows = pl.ds(sid * RPT, RPT)
    rb = (rb0, rb1, rb2)
    gs = (gsem0, gsem1, gsem2)

    pltpu.sync_copy(zh, acc.at[rows])
    plsc.subcore_barrier()

    def scatter_pass(idx_hbm, srcsel, tab, ngroups):
        dstsel = 1 - srcsel
        ih = idx_hbm.at[sid]

        def idxrow(c, which):
            buf = ia if c < G else ib
            return buf.at[c % G, which]

        pltpu.sync_copy(ih.at[pl.ds(0, G)], ia)
        pltpu.async_copy(ih.at[pl.ds(G, G)], ib, isem1)
        pltpu.async_copy(tab.at[ia.at[0, srcsel]], rb0, gsem0)
        pltpu.async_copy(tab.at[ia.at[1, srcsel]], rb1, gsem1)

        @pl.loop(0, ngroups // 2)
        def _(t):
            for c in range(PAIR):
                if c == G - 2:
                    pltpu.make_async_copy(ih.at[pl.ds(0, G)], ib, isem1).wait()
                b = c % 3
                pltpu.make_async_copy(tab.at[idxrow(c, srcsel)], rb[b],
                                      gs[b]).wait()
                if c + 2 < PAIR:
                    nb = (c + 2) % 3
                    pltpu.async_copy(tab.at[idxrow(c + 2, srcsel)], rb[nb],
                                     gs[nb])
                pltpu.sync_copy(rb[b], acc.at[idxrow(c, dstsel)], add=True)
                if c == G - 1:
                    g2 = jnp.minimum(2 * t + 2, ngroups - 1)
                    pltpu.async_copy(ih.at[pl.ds(g2 * G, G)], ia, isem0)
            g3 = jnp.minimum(2 * t + 3, ngroups - 1)
            pltpu.async_copy(ih.at[pl.ds(g3 * G, G)], ib, isem1)
            pltpu.make_async_copy(ih.at[pl.ds(0, G)], ia, isem0).wait()
            pltpu.async_copy(tab.at[ia.at[0, srcsel]], rb0, gsem0)
            pltpu.async_copy(tab.at[ia.at[1, srcsel]], rb1, gsem1)

        pltpu.make_async_copy(tab.at[ia.at[0, srcsel]], rb0, gsem0).wait()
        pltpu.make_async_copy(tab.at[ia.at[1, srcsel]], rb1, gsem1).wait()
        pltpu.make_async_copy(ih.at[pl.ds(0, G)], ib, isem1).wait()

    def core_work(rate_srcsel, rate_tab, rate_out, soc_blk, soc_out):
        scatter_pass(rcb, rate_srcsel, rate_tab, GR)
        plsc.subcore_barrier()
        pltpu.sync_copy(acc.at[rows], rate_out.at[rows])
        pltpu.sync_copy(zh, acc.at[rows])
        plsc.subcore_barrier()
        scatter_pass(soc_blk, 0, gu, GSH)
        plsc.subcore_barrier()
        pltpu.sync_copy(acc.at[rows], soc_out.at[rows])

    @pl.when(cid == 0)
    def _():
        core_work(1, hi, au_o, scb0, bu0_o)

    @pl.when(cid == 1)
    def _():
        core_work(0, hu, ai_o, scb1, bu1_o)


@functools.partial(
    pl.kernel,
    out_type=tuple(jax.ShapeDtypeStruct((32 * SCORE_PT, D), _f32)
                   for _ in range(2)),
    mesh=_mesh,
    compiler_params=pltpu.CompilerParams(use_tc_tiling_on_sc=False),
    scratch_types=[
        pltpu.VMEM((CP, CHUNK), _i32),
        pltpu.VMEM((CP, CHUNK), _i32),
        pltpu.VMEM((CHUNK, D), _f32),
        pltpu.VMEM((CHUNK, D), _f32),
        pltpu.VMEM((CHUNK, D), _f32),
        pltpu.VMEM((CHUNK, D), _f32),
        pltpu.SemaphoreType.DMA,
        pltpu.SemaphoreType.DMA,
        pltpu.SemaphoreType.DMA,
        pltpu.SemaphoreType.DMA,
    ],
)
def _score_prod_kernel(ru, ri, pub, pib, nub, nib,
                       pp, np_,
                       blku, blki, rbu0, rbi0, rbu1, rbi1,
                       su0, si0, su1, si1):
    cid = lax.axis_index("c")
    sid = lax.axis_index("s")
    wid = sid * 2 + cid
    base = wid * SCORE_PT

    def mul(bu, bi):
        @pl.loop(0, CHUNK)
        def _(r):
            for q in range(D // 16):
                sl = pl.ds(q * 16, 16)
                bu[r, sl] = bu[r, sl] * bi[r, sl]

    def one(ub_h, ib_h, out_h):
        pltpu.sync_copy(ub_h.at[wid], blku)
        pltpu.sync_copy(ib_h.at[wid], blki)
        pltpu.async_copy(ru.at[blku.at[0]], rbu0, su0)
        pltpu.async_copy(ri.at[blki.at[0]], rbi0, si0)

        @pl.loop(0, CP // 2)
        def _(t):
            a = 2 * t
            pltpu.make_async_copy(ru.at[blku.at[a]], rbu0, su0).wait()
            pltpu.make_async_copy(ri.at[blki.at[a]], rbi0, si0).wait()
            pltpu.async_copy(ru.at[blku.at[a + 1]], rbu1, su1)
            pltpu.async_copy(ri.at[blki.at[a + 1]], rbi1, si1)
            mul(rbu0, rbi0)
            pltpu.sync_copy(rbu0, out_h.at[pl.ds(base + a * CHUNK, CHUNK)])
            pltpu.make_async_copy(ru.at[blku.at[a + 1]], rbu1, su1).wait()
            pltpu.make_async_copy(ri.at[blki.at[a + 1]], rbi1, si1).wait()
            pltpu.async_copy(ru.at[blku.at[a + 2]], rbu0, su0)
            pltpu.async_copy(ri.at[blki.at[a + 2]], rbi0, si0)
            mul(rbu1, rbi1)
            pltpu.sync_copy(rbu1,
                            out_h.at[pl.ds(base + (a + 1) * CHUNK, CHUNK)])

        pltpu.make_async_copy(ru.at[blku.at[CP - 1]], rbu0, su0).wait()
        pltpu.make_async_copy(ri.at[blki.at[CP - 1]], rbi0, si0).wait()
        mul(rbu0, rbi0)
        pltpu.sync_copy(rbu0,
                        out_h.at[pl.ds(base + (CP - 1) * CHUNK, CHUNK)])

    one(pub, pib, pp)
    one(nub, nib, np_)


_DOT_BLOCK = 1024


def _dot_body(pp, np_, po, no):
    po[...] = jnp.sum(pp[...], axis=1, keepdims=True)
    no[...] = jnp.sum(np_[...], axis=1, keepdims=True)


_dot_kernel = pl.pallas_call(
    _dot_body,
    grid=(32 * SCORE_PT // _DOT_BLOCK,),
    in_specs=[pl.BlockSpec((_DOT_BLOCK, D), lambda b: (b, 0))] * 2,
    out_specs=[pl.BlockSpec((_DOT_BLOCK, 1), lambda b: (b, 0))] * 2,
    out_shape=[jax.ShapeDtypeStruct((32 * SCORE_PT, 1), _f32)] * 2,
)


def _edge_blocks(idx, per_tile, fill):
    idx = idx.astype(_i32)
    total = 16 * per_tile
    idx = jnp.pad(idx, (0, total - idx.shape[0]), constant_values=fill)
    return idx.reshape(16, per_tile // CHUNK, CHUNK)


def _score_blocks(idx):
    idx = idx.astype(_i32)
    idx = jnp.pad(idx, (0, 32 * SCORE_PT - idx.shape[0]))
    return idx.reshape(32, CP, CHUNK)


def kernel(user_emb, item_emb, edge_index_rate, edge_index_social,
           pos_edge_index, neg_edge_index):
    rcb = jnp.stack([_edge_blocks(edge_index_rate[0], RATE_PT, TRASH),
                     _edge_blocks(edge_index_rate[1], RATE_PT, TRASH)], axis=2)
    scb = jnp.stack([_edge_blocks(edge_index_social[0], SOC_PT, TRASH),
                     _edge_blocks(edge_index_social[1], SOC_PT, TRASH)], axis=2)
    scb0 = scb[:, :CSH]
    scb1 = scb[:, CSH:]

    e1 = jnp.zeros((CHUNK, 16), _f32).at[:, 0].set(1.0)
    zc = jnp.zeros((RPT, 16), _f32)
    zh = jnp.zeros((RPT, D), _f32)

    cru, cri, css, csd = _deg_kernel(rcb, scb, e1, zc)

    def scale(cnt):
        s = lax.rsqrt(jnp.clip(cnt[:N, 0], 1.0, None))
        return jnp.pad(s, (0, NPAD - N), constant_values=1.0)[:, None]

    au, bi, cs, ed = scale(cru), scale(cri), scale(css), scale(csd)

    cur_u = jnp.pad(user_emb, ((0, NPAD - N), (0, 0)))
    cur_i = jnp.pad(item_emb, ((0, NPAD - N), (0, 0)))
    res_u, res_i = cur_u, cur_i
    for _ in range(3):
        hu = au * cur_u
        hi = bi * cur_i
        gu = cs * cur_u
        Au, Ai, Bu0, Bu1 = _layer_kernel(hu, hi, gu, rcb, scb0, scb1, zh)
        emb_u = au * Au + ed * (Bu0 + Bu1)
        emb_i = bi * Ai
        res_u = res_u + emb_u
        res_i = res_i + emb_i
        cur_u, cur_i = emb_u, emb_i

    ru = res_u * 0.25
    ri = res_i * 0.25

    pub = _score_blocks(pos_edge_index[0])
    pib = _score_blocks(pos_edge_index[1])
    nub = _score_blocks(neg_edge_index[0])
    nib = _score_blocks(neg_edge_index[1])
    pp, np_ = _score_prod_kernel(ru, ri, pub, pib, nub, nib)
    pos_s, neg_s = _dot_kernel(pp, np_)
    return (pos_s[:P], neg_s[:P])
